# Initial kernel scaffold; baseline (speedup 1.0000x reference)
import jax
import jax.numpy as jnp
from jax.experimental import pallas as pl


def kernel(xyz, features, W0_0, b0_0, W0_1, b0_1, W1_0, b1_0, W1_1, b1_1, W2_0, b2_0, W2_1, b2_1):
    raise NotImplementedError("write your pallas kernel here")


# speedup vs baseline: 25.6245x; 25.6245x over previous
import functools

import jax
import jax.numpy as jnp
import numpy as np
from jax import lax
from jax.experimental import pallas as pl
from jax.experimental.pallas import tpu as pltpu
from jax.experimental.pallas import tpu_sc as plsc

B, N, C, D = 4, 4096, 3, 64
S = 1024
RADII2 = (0.2 ** 2, 0.4 ** 2, 0.8 ** 2)
KS = (16, 32, 64)
SCALE = float(1.0 / np.sqrt(1.0 + 1e-5))


def _fps_body(x_ref, y_ref, z_ref, nx_ref, ny_ref, nz_ref):
    Xv = x_ref[...]
    Yv = y_ref[...]
    Zv = z_ref[...]
    lane = lax.broadcasted_iota(jnp.int32, (B, N), 1)
    lane_s = lax.broadcasted_iota(jnp.int32, (B, S), 1)

    def body(i, carry):
        far, dist, ax, ay, az = carry
        onehot = lane == far
        cx = jnp.sum(jnp.where(onehot, Xv, 0.0), axis=1, keepdims=True)
        cy = jnp.sum(jnp.where(onehot, Yv, 0.0), axis=1, keepdims=True)
        cz = jnp.sum(jnp.where(onehot, Zv, 0.0), axis=1, keepdims=True)
        hit = lane_s == i
        ax = jnp.where(hit, cx, ax)
        ay = jnp.where(hit, cy, ay)
        az = jnp.where(hit, cz, az)
        dx = Xv - cx
        dy = Yv - cy
        dz = Zv - cz
        d = (dx * dx + dy * dy) + dz * dz
        dist = jnp.minimum(dist, d)
        m = jnp.max(dist, axis=1, keepdims=True)
        far = jnp.min(jnp.where(dist == m, lane, N), axis=1, keepdims=True)
        return far, dist, ax, ay, az

    far0 = jnp.zeros((B, 1), jnp.int32)
    dist0 = jnp.full((B, N), 1e10, jnp.float32)
    zs = jnp.zeros((B, S), jnp.float32)
    _, _, ax, ay, az = lax.fori_loop(0, S, body, (far0, dist0, zs, zs, zs))
    nx_ref[...] = ax
    ny_ref[...] = ay
    nz_ref[...] = az


def _fps(xc):
    out = pl.pallas_call(
        _fps_body,
        out_shape=[jax.ShapeDtypeStruct((B, S), jnp.float32)] * 3,
    )(xc[:, 0], xc[:, 1], xc[:, 2])
    return out


def _mlp_body(f_ref, w1_ref, b1_ref, w2_ref, b2_ref, y0_ref, y1_ref, y2_ref):
    x = f_ref[...]
    h = jnp.dot(x, w1_ref[...], preferred_element_type=jnp.float32)
    h = jnp.maximum((h + b1_ref[...]) * SCALE, 0.0)
    y = jnp.dot(h, w2_ref[...], preferred_element_type=jnp.float32)
    y = jnp.maximum((y + b2_ref[...]) * SCALE, 0.0)
    y0_ref[...] = y[:, 0:128]
    y1_ref[...] = y[:, 128:256]
    y2_ref[...] = y[:, 256:384]


def _mlp_all_points(features, w1, b1, w2, b2):
    rows = B * N
    blk = 2048
    grid = rows // blk
    outs = pl.pallas_call(
        _mlp_body,
        grid=(grid,),
        in_specs=[
            pl.BlockSpec((blk, 64), lambda i: (i, 0)),
            pl.BlockSpec((64, 288), lambda i: (0, 0)),
            pl.BlockSpec((1, 288), lambda i: (0, 0)),
            pl.BlockSpec((288, 384), lambda i: (0, 0)),
            pl.BlockSpec((1, 384), lambda i: (0, 0)),
        ],
        out_specs=[pl.BlockSpec((blk, 128), lambda i: (i, 0))] * 3,
        out_shape=[jax.ShapeDtypeStruct((rows, 128), jnp.float32)] * 3,
    )(features, w1, b1, w2, b2)
    return outs


_NC = 2
_NS = 16
_NW = _NC * _NS
_CPW = (B * S) // _NW


def _sc_body(px_hbm, py_hbm, pz_hbm, cx_hbm, cy_hbm, cz_hbm,
             y0_hbm, y1_hbm, y2_hbm, out_hbm,
             xv, yv, zv, cxv, cyv, czv,
             idx0, idx1, idx2, rows0, rows1, rows2, orow,
             sem0, sem1, sem2):
    wid = lax.axis_index("s") * _NC + lax.axis_index("c")
    b = wid // (_NW // B)
    pltpu.sync_copy(px_hbm.at[b], xv)
    pltpu.sync_copy(py_hbm.at[b], yv)
    pltpu.sync_copy(pz_hbm.at[b], zv)
    pltpu.sync_copy(cx_hbm.at[wid], cxv)
    pltpu.sync_copy(cy_hbm.at[wid], cyv)
    pltpu.sync_copy(cz_hbm.at[wid], czv)

    iota16 = lax.broadcasted_iota(jnp.int32, (16,), 0)
    z16 = jnp.zeros((16,), jnp.int32)
    bN = b * N
    base_row = wid * _CPW
    bufs = (idx0, idx1, idx2)
    rbufs = (rows0, rows1, rows2)
    sems = (sem0, sem1, sem2)

    def center(j, _):
        jv = jnp.full((16,), j, jnp.int32)
        cx = plsc.load_gather(cxv, [jv])
        cy = plsc.load_gather(cyv, [jv])
        cz = plsc.load_gather(czv, [jv])

        def cond(st):
            n, c0, c1, c2 = st
            return (n < N) & ((c0 < KS[0]) | (c1 < KS[1]) | (c2 < KS[2]))

        def scan(st):
            n, c0, c1, c2 = st
            nv = iota16 + n
            px = plsc.load_gather(xv, [nv])
            py = plsc.load_gather(yv, [nv])
            pz = plsc.load_gather(zv, [nv])
            dx = px - cx
            dy = py - cy
            dz = pz - cz
            d = (dx * dx + dy * dy) + dz * dz
            gv = nv + bN
            cnts = []
            for (cnt, K, r2, buf) in zip((c0, c1, c2), KS, RADII2, bufs):
                m = d <= r2
                csum = plsc.cumsum(m.astype(jnp.int32))
                pos = csum + (cnt - 1)
                plsc.store_scatter(buf, [pos], gv, mask=m & (pos < K))
                cnts.append(cnt + csum[15])
            return (n + 16, cnts[0], cnts[1], cnts[2])

        _, c0, c1, c2 = lax.while_loop(cond, scan, (0, 0, 0, 0))

        for (cnt, K, buf) in zip((c0, c1, c2), KS, bufs):
            i0 = plsc.load_gather(buf, [z16])
            for cc in range(K // 16):
                lanev = iota16 + cc * 16
                chunk = buf[cc * 16:(cc + 1) * 16]
                buf[cc * 16:(cc + 1) * 16] = jnp.where(lanev < cnt, chunk, i0)

        cps = [
            pltpu.async_copy(y_hbm.at[buf], rbuf, sem)
            for (y_hbm, buf, rbuf, sem) in zip(
                (y0_hbm, y1_hbm, y2_hbm), bufs, rbufs, sems)
        ]
        for bi, (cp, K, rbuf) in enumerate(zip(cps, KS, rbufs)):
            cp.wait()

            def red(k, acc):
                return tuple(
                    jnp.maximum(acc[cc], rbuf[k, cc * 16:(cc + 1) * 16])
                    for cc in range(8))

            acc = lax.fori_loop(
                0, K, red, tuple(jnp.full((16,), -jnp.inf, jnp.float32)
                                 for _ in range(8)))
            for cc in range(8):
                orow[bi * 128 + cc * 16: bi * 128 + (cc + 1) * 16] = acc[cc]
        pltpu.sync_copy(orow, out_hbm.at[base_row + j])
        return 0

    lax.fori_loop(0, _CPW, center, 0)


def _sc_group_max(px, py, pz, cx, cy, cz, y0, y1, y2):
    mesh = plsc.VectorSubcoreMesh(core_axis_name="c", subcore_axis_name="s")
    f = pl.kernel(
        _sc_body,
        out_type=jax.ShapeDtypeStruct((B * S, 384), jnp.float32),
        mesh=mesh,
        scratch_types=[
            pltpu.VMEM((N,), jnp.float32),
            pltpu.VMEM((N,), jnp.float32),
            pltpu.VMEM((N,), jnp.float32),
            pltpu.VMEM((_CPW,), jnp.float32),
            pltpu.VMEM((_CPW,), jnp.float32),
            pltpu.VMEM((_CPW,), jnp.float32),
            pltpu.VMEM((KS[0],), jnp.int32),
            pltpu.VMEM((KS[1],), jnp.int32),
            pltpu.VMEM((KS[2],), jnp.int32),
            pltpu.VMEM((KS[0], 128), jnp.float32),
            pltpu.VMEM((KS[1], 128), jnp.float32),
            pltpu.VMEM((KS[2], 128), jnp.float32),
            pltpu.VMEM((384,), jnp.float32),
            pltpu.SemaphoreType.DMA,
            pltpu.SemaphoreType.DMA,
            pltpu.SemaphoreType.DMA,
        ],
        compiler_params=pltpu.CompilerParams(needs_layout_passes=False),
    )
    return f(px, py, pz, cx, cy, cz, y0, y1, y2)


def kernel(xyz, features, W0_0, b0_0, W0_1, b0_1, W1_0, b1_0, W1_1, b1_1,
           W2_0, b2_0, W2_1, b2_1):
    xc = jnp.transpose(xyz, (0, 2, 1))
    nx, ny, nz = _fps(xc)
    new_xyz_t = jnp.stack([nx, ny, nz], axis=1)

    w1 = jnp.concatenate([W0_0.T, W1_0.T, W2_0.T], axis=1)
    b1 = jnp.concatenate([b0_0, b1_0, b2_0]).reshape(1, 288)
    w2 = jnp.zeros((288, 384), jnp.float32)
    w2 = w2.at[0:64, 0:128].set(W0_1.T)
    w2 = w2.at[64:160, 128:256].set(W1_1.T)
    w2 = w2.at[160:288, 256:384].set(W2_1.T)
    b2 = jnp.concatenate([b0_1, b1_1, b2_1]).reshape(1, 384)
    y0, y1, y2 = _mlp_all_points(features.reshape(B * N, D), w1, b1, w2, b2)

    cx = nx.reshape(_NW, _CPW)
    cy = ny.reshape(_NW, _CPW)
    cz = nz.reshape(_NW, _CPW)
    outp = _sc_group_max(xc[:, 0], xc[:, 1], xc[:, 2], cx, cy, cz,
                         y0, y1, y2)
    new_points = outp.reshape(B, S, 384).transpose(0, 2, 1)
    return new_xyz_t, new_points
